# Initial kernel scaffold; baseline (speedup 1.0000x reference)
import jax
import jax.numpy as jnp
from jax.experimental import pallas as pl


def kernel(hidden_states, router_w, w_gate, w_up, w_down, sh_gate, sh_up, sh_down):
    raise NotImplementedError("write your pallas kernel here")


# speedup vs baseline: 2.2751x; 2.2751x over previous
import functools

import jax
import jax.numpy as jnp
from jax.experimental import pallas as pl
from jax.experimental.pallas import tpu as pltpu

_B, _S, _D = 1, 2048, 1024
_E, _TOPK = 8, 2
_FF, _FF_SH = 256, 512
_T = _B * _S


def _silu(x):
    return x * (1.0 / (1.0 + jnp.exp(-x)))


def _moe_body(flat_ref, router_w_ref, wg_ref, wu_ref, wd_ref,
              shg_ref, shu_ref, shd_ref,
              out_ref, logits_ref, comb_ref):
    e = pl.program_id(0)
    flat = flat_ref[...]

    @pl.when(e == 0)
    def _prologue():
        logits = jnp.dot(flat, router_w_ref[...],
                         preferred_element_type=jnp.float32)
        logits_ref[...] = logits
        lmax = jnp.max(logits, axis=1, keepdims=True)
        p = jnp.exp(logits - lmax)
        m1 = jnp.max(p, axis=1, keepdims=True)
        lane = jax.lax.broadcasted_iota(jnp.int32, (_T, _E), 1)
        i1 = jnp.argmax(p, axis=1)[:, None]
        p2 = jnp.where(lane == i1, -jnp.inf, p)
        m2 = jnp.max(p2, axis=1, keepdims=True)
        i2 = jnp.argmax(p2, axis=1)[:, None]
        s = m1 + m2
        comb_ref[...] = ((lane == i1) * (m1 / s) + (lane == i2) * (m2 / s)
                         ).astype(jnp.float32)
        g = jnp.dot(flat, shg_ref[...], preferred_element_type=jnp.float32)
        u = jnp.dot(flat, shu_ref[...], preferred_element_type=jnp.float32)
        out_ref[...] = jnp.dot(_silu(g) * u, shd_ref[...],
                               preferred_element_type=jnp.float32)

    lane = jax.lax.broadcasted_iota(jnp.int32, (_T, _E), 1)
    comb_col = jnp.sum(jnp.where(lane == e, comb_ref[...], 0.0), axis=1,
                       keepdims=True)
    g = jnp.dot(flat, wg_ref[0], preferred_element_type=jnp.float32)
    u = jnp.dot(flat, wu_ref[0], preferred_element_type=jnp.float32)
    h = _silu(g) * u
    out_ref[...] += comb_col * jnp.dot(h, wd_ref[0],
                                       preferred_element_type=jnp.float32)


@functools.partial(jax.jit, static_argnames=("interpret",))
def kernel(hidden_states, router_w, w_gate, w_up, w_down,
           sh_gate, sh_up, sh_down, interpret=False):
    flat = hidden_states.reshape(_T, _D)
    out, logits = pl.pallas_call(
        _moe_body,
        grid=(_E,),
        in_specs=[
            pl.BlockSpec((_T, _D), lambda e: (0, 0)),
            pl.BlockSpec((_D, _E), lambda e: (0, 0)),
            pl.BlockSpec((1, _D, _FF), lambda e: (e, 0, 0)),
            pl.BlockSpec((1, _D, _FF), lambda e: (e, 0, 0)),
            pl.BlockSpec((1, _FF, _D), lambda e: (e, 0, 0)),
            pl.BlockSpec((_D, _FF_SH), lambda e: (0, 0)),
            pl.BlockSpec((_D, _FF_SH), lambda e: (0, 0)),
            pl.BlockSpec((_FF_SH, _D), lambda e: (0, 0)),
        ],
        out_specs=[
            pl.BlockSpec((_T, _D), lambda e: (0, 0)),
            pl.BlockSpec((_T, _E), lambda e: (0, 0)),
        ],
        out_shape=[
            jax.ShapeDtypeStruct((_T, _D), jnp.float32),
            jax.ShapeDtypeStruct((_T, _E), jnp.float32),
        ],
        scratch_shapes=[pltpu.VMEM((_T, _E), jnp.float32)],
        compiler_params=pltpu.CompilerParams(
            dimension_semantics=("arbitrary",),
        ),
        interpret=interpret,
    )(flat, router_w, w_gate, w_up, w_down, sh_gate, sh_up, sh_down)
    return out.reshape(_B, _S, _D), logits
